# baseline (speedup 1.0000x reference)
import functools

import jax
import jax.numpy as jnp
from jax import lax
from jax.experimental import pallas as pl
from jax.experimental.pallas import tpu as pltpu
from jax.experimental.pallas import tpu_sc as plsc

N = 10000
E = 320000
D_IN = 128
D_H = 128
D_OUT = 64
BN_EPS = 1e-5

NC = 2
NS = 16
NW = NC * NS
EPW = E // NW
K = 128
NB = 2
EPWP = 10240
NCHUNK = EPWP // K
NHALF = 2
HCH = NCHUNK // NHALF
RA = 624
DEGW = 16

BR = 400
GRID = N // BR


def _sc_mesh():
    return plsc.VectorSubcoreMesh(core_axis_name="c", subcore_axis_name="s")


def _slab_copy(s, src_of, dst_of):
    pltpu.sync_copy(src_of(s * RA, RA), dst_of(s * RA, RA))

    @pl.when(s == NS - 1)
    def _tail():
        pltpu.sync_copy(src_of(NS * RA, N - NS * RA),
                        dst_of(NS * RA, N - NS * RA))


NP = 10240
SLICE = NP // NS


@functools.lru_cache(maxsize=None)
def _build_degree():
    @functools.partial(
        pl.kernel,
        out_type=jax.ShapeDtypeStruct((NC, 1, NP), jnp.float32),
        mesh=_sc_mesh(),
        compiler_params=pltpu.CompilerParams(needs_layout_passes=False),
        scratch_types=[
            pltpu.VMEM((EPW,), jnp.int32),
            pltpu.VMEM((NP,), jnp.float32),
            pltpu.VMEM((NS, SLICE), jnp.float32),
            pltpu.VMEM_SHARED((NS, NP), jnp.float32),
        ],
    )
    def deg_kernel(dst_hbm, out_hbm, didx, hist, red, shared):
        c = lax.axis_index("c")
        s = lax.axis_index("s")
        w = c * NS + s
        zeros16 = jnp.zeros((16,), jnp.float32)
        ones16 = jnp.ones((16,), jnp.float32)

        @pl.loop(0, NP // 16)
        def _zero(i):
            hist[pl.ds(i * 16, 16)] = zeros16

        pltpu.sync_copy(dst_hbm.at[pl.ds(w * EPW, EPW)], didx)

        @pl.loop(0, EPW // 16)
        def _hist(j):
            v = didx[pl.ds(j * 16, 16)]
            plsc.addupdate_scatter(hist, [v], ones16)

        pltpu.sync_copy(hist, shared.at[s])
        plsc.subcore_barrier()
        pltpu.sync_copy(shared.at[:, pl.ds(s * SLICE, SLICE)], red)

        @pl.loop(0, SLICE // 16)
        def _red(j):
            acc = red[0, pl.ds(j * 16, 16)]
            for r in range(1, NS):
                acc = acc + red[r, pl.ds(j * 16, 16)]
            hist[pl.ds(j * 16, 16)] = acc

        pltpu.sync_copy(hist.at[pl.ds(0, SLICE)],
                        out_hbm.at[c, 0, pl.ds(s * SLICE, SLICE)])

    return deg_kernel


@functools.lru_cache(maxsize=None)
def _build_aggregate(d):
    rows_scratch = [pltpu.VMEM((K, d), jnp.float32) for _ in range(NB)]
    sem_scratch = [pltpu.SemaphoreType.DMA for _ in range(2 * NB)]

    @functools.partial(
        pl.kernel,
        out_type=jax.ShapeDtypeStruct((NC, N, d), jnp.float32),
        mesh=_sc_mesh(),
        scratch_types=[
            pltpu.VMEM((HCH, K), jnp.int32),
            pltpu.VMEM((HCH, K), jnp.int32),
            pltpu.VMEM_SHARED((N, d), jnp.float32),
        ] + rows_scratch + sem_scratch,
    )
    def agg_kernel(y_hbm, src3_hbm, dst3_hbm, zeros_hbm, out_hbm,
                   sidx_l, didx_l, acc, *bufs):
        rows = bufs[:NB]
        gsem = bufs[NB:2 * NB]
        ssem = bufs[2 * NB:]
        c = lax.axis_index("c")
        s = lax.axis_index("s")
        w = c * NS + s

        _slab_copy(s, lambda o, n: zeros_hbm.at[pl.ds(o, n)],
                   lambda o, n: acc.at[pl.ds(o, n)])
        plsc.subcore_barrier()

        @pl.loop(0, NHALF)
        def _half(h):
            pltpu.sync_copy(src3_hbm.at[w, pl.ds(h * HCH, HCH)], sidx_l)
            pltpu.sync_copy(dst3_hbm.at[w, pl.ds(h * HCH, HCH)], didx_l)

            @pl.loop(0, HCH, step=NB)
            def _edges(i0):
                gs = [pltpu.async_copy(y_hbm.at[sidx_l.at[i0 + b]], rows[b],
                                       gsem[b]) for b in range(NB)]
                ss = []
                for b in range(NB):
                    gs[b].wait()
                    ss.append(pltpu.async_copy(rows[b],
                                               acc.at[didx_l.at[i0 + b]],
                                               ssem[b], add=True))
                for desc in ss:
                    desc.wait()

        plsc.subcore_barrier()
        _slab_copy(s, lambda o, n: acc.at[pl.ds(o, n)],
                   lambda o, n: out_hbm.at[c, pl.ds(o, n)])

    return agg_kernel


def _tc1_body(x_ref, w_ref, deg_ref, y_ref, dinv_ref):
    indeg = deg_ref[:, 0:1] + deg_ref[:, 1:2]
    dinv = lax.rsqrt(1.0 + indeg)
    xw = jnp.dot(x_ref[...], w_ref[...], preferred_element_type=jnp.float32)
    y_ref[...] = xw * dinv
    dinv_ref[...] = dinv


@functools.lru_cache(maxsize=None)
def _build_tc1():
    return pl.pallas_call(
        _tc1_body,
        grid=(GRID,),
        in_specs=[
            pl.BlockSpec((BR, D_IN), lambda i: (i, 0)),
            pl.BlockSpec((D_IN, D_H), lambda i: (0, 0)),
            pl.BlockSpec((BR, NC), lambda i: (i, 0)),
        ],
        out_specs=[
            pl.BlockSpec((BR, D_H), lambda i: (i, 0)),
            pl.BlockSpec((BR, 1), lambda i: (i, 0)),
        ],
        out_shape=[
            jax.ShapeDtypeStruct((N, D_H), jnp.float32),
            jax.ShapeDtypeStruct((N, 1), jnp.float32),
        ],
    )


def _tcmid_body(agg_ref, y_ref, dinv_ref, b_ref, g_ref, be_ref, w_ref,
                out_ref):
    dinv = dinv_ref[...]
    z = dinv * (agg_ref[0] + agg_ref[1] + y_ref[...]) + b_ref[...]
    bn_scale = 1.0 / (1.0 + BN_EPS) ** 0.5
    h = jnp.maximum(g_ref[...] * (z * bn_scale) + be_ref[...], 0.0)
    out_ref[...] = dinv * jnp.dot(h, w_ref[...],
                                  preferred_element_type=jnp.float32)


@functools.lru_cache(maxsize=None)
def _build_tcmid(d_out):
    return pl.pallas_call(
        _tcmid_body,
        grid=(GRID,),
        in_specs=[
            pl.BlockSpec((NC, BR, D_H), lambda i: (0, i, 0)),
            pl.BlockSpec((BR, D_H), lambda i: (i, 0)),
            pl.BlockSpec((BR, 1), lambda i: (i, 0)),
            pl.BlockSpec((1, D_H), lambda i: (0, 0)),
            pl.BlockSpec((1, D_H), lambda i: (0, 0)),
            pl.BlockSpec((1, D_H), lambda i: (0, 0)),
            pl.BlockSpec((D_H, d_out), lambda i: (0, 0)),
        ],
        out_specs=pl.BlockSpec((BR, d_out), lambda i: (i, 0)),
        out_shape=jax.ShapeDtypeStruct((N, d_out), jnp.float32),
    )


def _tcf_body(agg_ref, y_ref, dinv_ref, b_ref, out_ref):
    zf = dinv_ref[...] * (agg_ref[0] + agg_ref[1] + y_ref[...])
    z = zf[:, :D_OUT] + b_ref[...]
    m = jnp.max(z, axis=1, keepdims=True)
    ez = jnp.exp(z - m)
    out_ref[...] = (z - m) - jnp.log(jnp.sum(ez, axis=1, keepdims=True))


@functools.lru_cache(maxsize=None)
def _build_tcf():
    return pl.pallas_call(
        _tcf_body,
        grid=(GRID,),
        in_specs=[
            pl.BlockSpec((NC, BR, D_H), lambda i: (0, i, 0)),
            pl.BlockSpec((BR, D_H), lambda i: (i, 0)),
            pl.BlockSpec((BR, 1), lambda i: (i, 0)),
            pl.BlockSpec((1, D_OUT), lambda i: (0, 0)),
        ],
        out_specs=pl.BlockSpec((BR, D_OUT), lambda i: (i, 0)),
        out_shape=jax.ShapeDtypeStruct((N, D_OUT), jnp.float32),
    )


def kernel(x, edge_index, W1, b1, g1, be1, W2, b2, g2, be2, W3, b3):
    src = edge_index[0]
    dst = edge_index[1]
    npad = NW * EPWP - E
    srcp = jnp.concatenate([src, jnp.full((npad,), N, src.dtype)])
    dstp = jnp.concatenate(
        [dst, jnp.arange(npad, dtype=dst.dtype) % jnp.int32(N)])
    src3 = srcp.reshape(NW, NCHUNK, K)
    dst3 = dstp.reshape(NW, NCHUNK, K)
    zeros_h = jnp.zeros((N, D_H), jnp.float32)
    ztail = jnp.zeros((8, D_H), jnp.float32)
    W3p = jnp.pad(W3, ((0, 0), (0, D_H - D_OUT)))

    degp = _build_degree()(dst)
    degT = degp.reshape(NC, NP)[:, :N].T
    y1, dinv = _build_tc1()(x, W1, degT)

    agg1 = _build_aggregate(D_H)(jnp.concatenate([y1, ztail]), src3, dst3,
                                 zeros_h)
    y2 = _build_tcmid(D_H)(agg1, y1, dinv, b1.reshape(1, -1),
                           g1.reshape(1, -1), be1.reshape(1, -1), W2)

    agg2 = _build_aggregate(D_H)(jnp.concatenate([y2, ztail]), src3, dst3,
                                 zeros_h)
    y3 = _build_tcmid(D_H)(agg2, y2, dinv, b2.reshape(1, -1),
                           g2.reshape(1, -1), be2.reshape(1, -1), W3p)

    agg3 = _build_aggregate(D_H)(jnp.concatenate([y3, ztail]), src3, dst3,
                                 zeros_h)
    return _build_tcf()(agg3, y3, dinv, b3.reshape(1, -1))


# speedup vs baseline: 2.6945x; 2.6945x over previous
import functools

import jax
import jax.numpy as jnp
from jax import lax
from jax.experimental import pallas as pl
from jax.experimental.pallas import tpu as pltpu
from jax.experimental.pallas import tpu_sc as plsc

N = 10000
E = 320000
D_IN = 128
D_H = 128
D_OUT = 64
BN_EPS = 1e-5

NC = 2
NS = 16
NW = NC * NS
EPW = E // NW
K = 128
NB = 1
EPWP = 10240
NCHUNK = EPWP // K
NHALF = 2
HCH = NCHUNK // NHALF
RA = 624
DEGW = 16

BR = 400
GRID = N // BR


def _sc_mesh():
    return plsc.VectorSubcoreMesh(core_axis_name="c", subcore_axis_name="s")


def _slab_copy(s, src_of, dst_of):
    pltpu.sync_copy(src_of(s * RA, RA), dst_of(s * RA, RA))

    @pl.when(s == NS - 1)
    def _tail():
        pltpu.sync_copy(src_of(NS * RA, N - NS * RA),
                        dst_of(NS * RA, N - NS * RA))


NP = 10240
SLICE = NP // NS


@functools.lru_cache(maxsize=None)
def _build_degree():
    @functools.partial(
        pl.kernel,
        out_type=jax.ShapeDtypeStruct((NC, 1, NP), jnp.float32),
        mesh=_sc_mesh(),
        compiler_params=pltpu.CompilerParams(needs_layout_passes=False),
        scratch_types=[
            pltpu.VMEM((EPW,), jnp.int32),
            pltpu.VMEM((NP,), jnp.float32),
            pltpu.VMEM((NS, SLICE), jnp.float32),
            pltpu.VMEM_SHARED((NS, NP), jnp.float32),
        ],
    )
    def deg_kernel(dst_hbm, out_hbm, didx, hist, red, shared):
        c = lax.axis_index("c")
        s = lax.axis_index("s")
        w = c * NS + s
        zeros16 = jnp.zeros((16,), jnp.float32)
        ones16 = jnp.ones((16,), jnp.float32)

        @pl.loop(0, NP // 16)
        def _zero(i):
            hist[pl.ds(i * 16, 16)] = zeros16

        pltpu.sync_copy(dst_hbm.at[pl.ds(w * EPW, EPW)], didx)

        @pl.loop(0, EPW // 16)
        def _hist(j):
            v = didx[pl.ds(j * 16, 16)]
            plsc.addupdate_scatter(hist, [v], ones16)

        pltpu.sync_copy(hist, shared.at[s])
        plsc.subcore_barrier()
        pltpu.sync_copy(shared.at[:, pl.ds(s * SLICE, SLICE)], red)

        @pl.loop(0, SLICE // 16)
        def _red(j):
            acc = red[0, pl.ds(j * 16, 16)]
            for r in range(1, NS):
                acc = acc + red[r, pl.ds(j * 16, 16)]
            hist[pl.ds(j * 16, 16)] = acc

        pltpu.sync_copy(hist.at[pl.ds(0, SLICE)],
                        out_hbm.at[c, 0, pl.ds(s * SLICE, SLICE)])

    return deg_kernel


KA = 80
NCHA = EPW // KA
NBUF = 4
NGRP = (NCHA - 1) // NBUF * NBUF


@functools.lru_cache(maxsize=None)
def _build_aggregate(d):
    scratch = (
        [pltpu.VMEM((KA,), jnp.int32) for _ in range(2 * NBUF)]
        + [pltpu.VMEM((KA, d), jnp.float32) for _ in range(NBUF)]
        + [pltpu.SemaphoreType.DMA for _ in range(2 * NBUF)]
        + [pltpu.VMEM_SHARED((N, d), jnp.float32)]
    )

    @functools.partial(
        pl.kernel,
        out_type=jax.ShapeDtypeStruct((NC, N, d), jnp.float32),
        mesh=_sc_mesh(),
        scratch_types=scratch,
    )
    def agg_kernel(y_hbm, src_hbm, dst_hbm, zeros_hbm, out_hbm, *bufs):
        sidx = bufs[:NBUF]
        didx = bufs[NBUF:2 * NBUF]
        rows = bufs[2 * NBUF:3 * NBUF]
        gsem = bufs[3 * NBUF:4 * NBUF]
        ssem = bufs[4 * NBUF:5 * NBUF]
        acc = bufs[5 * NBUF]
        c = lax.axis_index("c")
        s = lax.axis_index("s")
        w = c * NS + s

        _slab_copy(s, lambda o, n: zeros_hbm.at[pl.ds(o, n)],
                   lambda o, n: acc.at[pl.ds(o, n)])
        plsc.subcore_barrier()

        @pl.loop(0, NGRP, step=NBUF)
        def _edges(i0):
            gs = []
            for b in range(NBUF):
                base = w * EPW + (i0 + b) * KA
                pltpu.sync_copy(src_hbm.at[pl.ds(base, KA)], sidx[b])
                pltpu.sync_copy(dst_hbm.at[pl.ds(base, KA)], didx[b])
                gs.append(pltpu.async_copy(y_hbm.at[sidx[b]], rows[b],
                                           gsem[b]))
            ss = []
            for b in range(NBUF):
                gs[b].wait()
                ss.append(pltpu.async_copy(rows[b], acc.at[didx[b]],
                                           ssem[b], add=True))
            for desc in ss:
                desc.wait()

        for t in range(NGRP, NCHA):
            tbase = w * EPW + t * KA
            pltpu.sync_copy(src_hbm.at[pl.ds(tbase, KA)], sidx[0])
            pltpu.sync_copy(dst_hbm.at[pl.ds(tbase, KA)], didx[0])
            pltpu.async_copy(y_hbm.at[sidx[0]], rows[0], gsem[0]).wait()
            pltpu.sync_copy(rows[0], acc.at[didx[0]], add=True)

        plsc.subcore_barrier()
        _slab_copy(s, lambda o, n: acc.at[pl.ds(o, n)],
                   lambda o, n: out_hbm.at[c, pl.ds(o, n)])

    return agg_kernel


def _tc1_body(x_ref, w_ref, deg_ref, y_ref, dinv_ref):
    indeg = deg_ref[:, 0:1] + deg_ref[:, 1:2]
    dinv = lax.rsqrt(1.0 + indeg)
    xw = jnp.dot(x_ref[...], w_ref[...], preferred_element_type=jnp.float32)
    y_ref[...] = xw * dinv
    dinv_ref[...] = dinv


@functools.lru_cache(maxsize=None)
def _build_tc1():
    return pl.pallas_call(
        _tc1_body,
        grid=(GRID,),
        in_specs=[
            pl.BlockSpec((BR, D_IN), lambda i: (i, 0)),
            pl.BlockSpec((D_IN, D_H), lambda i: (0, 0)),
            pl.BlockSpec((BR, NC), lambda i: (i, 0)),
        ],
        out_specs=[
            pl.BlockSpec((BR, D_H), lambda i: (i, 0)),
            pl.BlockSpec((BR, 1), lambda i: (i, 0)),
        ],
        out_shape=[
            jax.ShapeDtypeStruct((N, D_H), jnp.float32),
            jax.ShapeDtypeStruct((N, 1), jnp.float32),
        ],
    )


def _tcmid_body(agg_ref, y_ref, dinv_ref, b_ref, g_ref, be_ref, w_ref,
                out_ref):
    dinv = dinv_ref[...]
    z = dinv * (agg_ref[0] + agg_ref[1] + y_ref[...]) + b_ref[...]
    bn_scale = 1.0 / (1.0 + BN_EPS) ** 0.5
    h = jnp.maximum(g_ref[...] * (z * bn_scale) + be_ref[...], 0.0)
    out_ref[...] = dinv * jnp.dot(h, w_ref[...],
                                  preferred_element_type=jnp.float32)


@functools.lru_cache(maxsize=None)
def _build_tcmid(d_out):
    return pl.pallas_call(
        _tcmid_body,
        grid=(GRID,),
        in_specs=[
            pl.BlockSpec((NC, BR, D_H), lambda i: (0, i, 0)),
            pl.BlockSpec((BR, D_H), lambda i: (i, 0)),
            pl.BlockSpec((BR, 1), lambda i: (i, 0)),
            pl.BlockSpec((1, D_H), lambda i: (0, 0)),
            pl.BlockSpec((1, D_H), lambda i: (0, 0)),
            pl.BlockSpec((1, D_H), lambda i: (0, 0)),
            pl.BlockSpec((D_H, d_out), lambda i: (0, 0)),
        ],
        out_specs=pl.BlockSpec((BR, d_out), lambda i: (i, 0)),
        out_shape=jax.ShapeDtypeStruct((N, d_out), jnp.float32),
    )


def _tcf_body(agg_ref, y_ref, dinv_ref, b_ref, out_ref):
    zf = dinv_ref[...] * (agg_ref[0] + agg_ref[1] + y_ref[...])
    z = zf[:, :D_OUT] + b_ref[...]
    m = jnp.max(z, axis=1, keepdims=True)
    ez = jnp.exp(z - m)
    out_ref[...] = (z - m) - jnp.log(jnp.sum(ez, axis=1, keepdims=True))


@functools.lru_cache(maxsize=None)
def _build_tcf():
    return pl.pallas_call(
        _tcf_body,
        grid=(GRID,),
        in_specs=[
            pl.BlockSpec((NC, BR, D_H), lambda i: (0, i, 0)),
            pl.BlockSpec((BR, D_H), lambda i: (i, 0)),
            pl.BlockSpec((BR, 1), lambda i: (i, 0)),
            pl.BlockSpec((1, D_OUT), lambda i: (0, 0)),
        ],
        out_specs=pl.BlockSpec((BR, D_OUT), lambda i: (i, 0)),
        out_shape=jax.ShapeDtypeStruct((N, D_OUT), jnp.float32),
    )


def kernel(x, edge_index, W1, b1, g1, be1, W2, b2, g2, be2, W3, b3):
    src = edge_index[0]
    dst = edge_index[1]
    zeros_h = jnp.zeros((N, D_H), jnp.float32)
    W3p = jnp.pad(W3, ((0, 0), (0, D_H - D_OUT)))

    degp = _build_degree()(dst)
    degT = degp.reshape(NC, NP)[:, :N].T
    y1, dinv = _build_tc1()(x, W1, degT)

    agg1 = _build_aggregate(D_H)(y1, src, dst, zeros_h)
    y2 = _build_tcmid(D_H)(agg1, y1, dinv, b1.reshape(1, -1),
                           g1.reshape(1, -1), be1.reshape(1, -1), W2)

    agg2 = _build_aggregate(D_H)(y2, src, dst, zeros_h)
    y3 = _build_tcmid(D_H)(agg2, y2, dinv, b2.reshape(1, -1),
                           g2.reshape(1, -1), be2.reshape(1, -1), W3p)

    agg3 = _build_aggregate(D_H)(y3, src, dst, zeros_h)
    return _build_tcf()(agg3, y3, dinv, b3.reshape(1, -1))


# speedup vs baseline: 3.0236x; 1.1221x over previous
import functools

import jax
import jax.numpy as jnp
from jax import lax
from jax.experimental import pallas as pl
from jax.experimental.pallas import tpu as pltpu
from jax.experimental.pallas import tpu_sc as plsc

N = 10000
E = 320000
D_IN = 128
D_H = 128
D_OUT = 64
BN_EPS = 1e-5

NC = 2
NS = 16
NW = NC * NS
EPW = E // NW
K = 128
NB = 1
EPWP = 10240
NCHUNK = EPWP // K
NHALF = 2
HCH = NCHUNK // NHALF
RA = 624
DEGW = 16

BR = 400
GRID = N // BR


def _sc_mesh():
    return plsc.VectorSubcoreMesh(core_axis_name="c", subcore_axis_name="s")


def _slab_copy(s, src_of, dst_of):
    pltpu.sync_copy(src_of(s * RA, RA), dst_of(s * RA, RA))

    @pl.when(s == NS - 1)
    def _tail():
        pltpu.sync_copy(src_of(NS * RA, N - NS * RA),
                        dst_of(NS * RA, N - NS * RA))


NP = 10240
SLICE = NP // NS


@functools.lru_cache(maxsize=None)
def _build_degree():
    @functools.partial(
        pl.kernel,
        out_type=jax.ShapeDtypeStruct((NC, 1, NP), jnp.float32),
        mesh=_sc_mesh(),
        compiler_params=pltpu.CompilerParams(needs_layout_passes=False),
        scratch_types=[
            pltpu.VMEM((EPW,), jnp.int32),
            pltpu.VMEM((NP,), jnp.float32),
            pltpu.VMEM((NS, SLICE), jnp.float32),
            pltpu.VMEM_SHARED((NS, NP), jnp.float32),
        ],
    )
    def deg_kernel(dst_hbm, out_hbm, didx, hist, red, shared):
        c = lax.axis_index("c")
        s = lax.axis_index("s")
        w = c * NS + s
        zeros16 = jnp.zeros((16,), jnp.float32)
        ones16 = jnp.ones((16,), jnp.float32)

        @pl.loop(0, NP // 16)
        def _zero(i):
            hist[pl.ds(i * 16, 16)] = zeros16

        pltpu.sync_copy(dst_hbm.at[pl.ds(w * EPW, EPW)], didx)

        @pl.loop(0, EPW // 16)
        def _hist(j):
            v = didx[pl.ds(j * 16, 16)]
            plsc.addupdate_scatter(hist, [v], ones16)

        pltpu.sync_copy(hist, shared.at[s])
        plsc.subcore_barrier()
        pltpu.sync_copy(shared.at[:, pl.ds(s * SLICE, SLICE)], red)

        @pl.loop(0, SLICE // 16)
        def _red(j):
            acc = red[0, pl.ds(j * 16, 16)]
            for r in range(1, NS):
                acc = acc + red[r, pl.ds(j * 16, 16)]
            hist[pl.ds(j * 16, 16)] = acc

        pltpu.sync_copy(hist.at[pl.ds(0, SLICE)],
                        out_hbm.at[c, 0, pl.ds(s * SLICE, SLICE)])

    return deg_kernel


KA = 80
NCHA = EPW // KA
NBUF = 4
NGRP = (NCHA - 1) // NBUF * NBUF


@functools.lru_cache(maxsize=None)
def _build_aggregate(d):
    scratch = (
        [pltpu.VMEM((KA,), jnp.int32) for _ in range(2 * NBUF)]
        + [pltpu.VMEM((KA, d), jnp.float32) for _ in range(NBUF)]
        + [pltpu.SemaphoreType.DMA for _ in range(4 * NBUF)]
        + [pltpu.VMEM_SHARED((N, d), jnp.float32)]
    )

    @functools.partial(
        pl.kernel,
        out_type=jax.ShapeDtypeStruct((NC, N, d), jnp.float32),
        mesh=_sc_mesh(),
        scratch_types=scratch,
    )
    def agg_kernel(y_hbm, src_hbm, dst_hbm, zeros_hbm, out_hbm, *bufs):
        sidx = bufs[:NBUF]
        didx = bufs[NBUF:2 * NBUF]
        rows = bufs[2 * NBUF:3 * NBUF]
        gsem = bufs[3 * NBUF:4 * NBUF]
        ssem = bufs[4 * NBUF:5 * NBUF]
        isem = bufs[5 * NBUF:6 * NBUF]
        jsem = bufs[6 * NBUF:7 * NBUF]
        acc = bufs[7 * NBUF]
        c = lax.axis_index("c")
        s = lax.axis_index("s")
        w = c * NS + s

        _slab_copy(s, lambda o, n: zeros_hbm.at[pl.ds(o, n)],
                   lambda o, n: acc.at[pl.ds(o, n)])
        plsc.subcore_barrier()

        @pl.loop(0, NGRP, step=NBUF)
        def _edges(i0):
            @pl.when(i0 > 0)
            def _drain_prev():
                for b in range(NBUF):
                    pltpu.make_async_copy(rows[b], acc.at[didx[b]],
                                          ssem[b]).wait()

            ils, jls = [], []
            for b in range(NBUF):
                base = w * EPW + (i0 + b) * KA
                ils.append(pltpu.async_copy(src_hbm.at[pl.ds(base, KA)],
                                            sidx[b], isem[b]))
                jls.append(pltpu.async_copy(dst_hbm.at[pl.ds(base, KA)],
                                            didx[b], jsem[b]))
            gs = []
            for b in range(NBUF):
                ils[b].wait()
                gs.append(pltpu.async_copy(y_hbm.at[sidx[b]], rows[b],
                                           gsem[b]))
            for b in range(NBUF):
                gs[b].wait()
                jls[b].wait()
                pltpu.async_copy(rows[b], acc.at[didx[b]], ssem[b], add=True)

        for b in range(NBUF):
            pltpu.make_async_copy(rows[b], acc.at[didx[b]], ssem[b]).wait()

        for t in range(NGRP, NCHA):
            tbase = w * EPW + t * KA
            pltpu.sync_copy(src_hbm.at[pl.ds(tbase, KA)], sidx[0])
            pltpu.sync_copy(dst_hbm.at[pl.ds(tbase, KA)], didx[0])
            pltpu.async_copy(y_hbm.at[sidx[0]], rows[0], gsem[0]).wait()
            pltpu.sync_copy(rows[0], acc.at[didx[0]], add=True)

        plsc.subcore_barrier()
        _slab_copy(s, lambda o, n: acc.at[pl.ds(o, n)],
                   lambda o, n: out_hbm.at[c, pl.ds(o, n)])

    return agg_kernel


def _tc1_body(x_ref, w_ref, deg_ref, y_ref, dinv_ref):
    indeg = deg_ref[:, 0:1] + deg_ref[:, 1:2]
    dinv = lax.rsqrt(1.0 + indeg)
    xw = jnp.dot(x_ref[...], w_ref[...], preferred_element_type=jnp.float32)
    y_ref[...] = xw * dinv
    dinv_ref[...] = dinv


@functools.lru_cache(maxsize=None)
def _build_tc1():
    return pl.pallas_call(
        _tc1_body,
        grid=(GRID,),
        in_specs=[
            pl.BlockSpec((BR, D_IN), lambda i: (i, 0)),
            pl.BlockSpec((D_IN, D_H), lambda i: (0, 0)),
            pl.BlockSpec((BR, NC), lambda i: (i, 0)),
        ],
        out_specs=[
            pl.BlockSpec((BR, D_H), lambda i: (i, 0)),
            pl.BlockSpec((BR, 1), lambda i: (i, 0)),
        ],
        out_shape=[
            jax.ShapeDtypeStruct((N, D_H), jnp.float32),
            jax.ShapeDtypeStruct((N, 1), jnp.float32),
        ],
    )


def _tcmid_body(agg_ref, y_ref, dinv_ref, b_ref, g_ref, be_ref, w_ref,
                out_ref):
    dinv = dinv_ref[...]
    z = dinv * (agg_ref[0] + agg_ref[1] + y_ref[...]) + b_ref[...]
    bn_scale = 1.0 / (1.0 + BN_EPS) ** 0.5
    h = jnp.maximum(g_ref[...] * (z * bn_scale) + be_ref[...], 0.0)
    out_ref[...] = dinv * jnp.dot(h, w_ref[...],
                                  preferred_element_type=jnp.float32)


@functools.lru_cache(maxsize=None)
def _build_tcmid(d_out):
    return pl.pallas_call(
        _tcmid_body,
        grid=(GRID,),
        in_specs=[
            pl.BlockSpec((NC, BR, D_H), lambda i: (0, i, 0)),
            pl.BlockSpec((BR, D_H), lambda i: (i, 0)),
            pl.BlockSpec((BR, 1), lambda i: (i, 0)),
            pl.BlockSpec((1, D_H), lambda i: (0, 0)),
            pl.BlockSpec((1, D_H), lambda i: (0, 0)),
            pl.BlockSpec((1, D_H), lambda i: (0, 0)),
            pl.BlockSpec((D_H, d_out), lambda i: (0, 0)),
        ],
        out_specs=pl.BlockSpec((BR, d_out), lambda i: (i, 0)),
        out_shape=jax.ShapeDtypeStruct((N, d_out), jnp.float32),
    )


def _tcf_body(agg_ref, y_ref, dinv_ref, b_ref, out_ref):
    zf = dinv_ref[...] * (agg_ref[0] + agg_ref[1] + y_ref[...])
    z = zf[:, :D_OUT] + b_ref[...]
    m = jnp.max(z, axis=1, keepdims=True)
    ez = jnp.exp(z - m)
    out_ref[...] = (z - m) - jnp.log(jnp.sum(ez, axis=1, keepdims=True))


@functools.lru_cache(maxsize=None)
def _build_tcf():
    return pl.pallas_call(
        _tcf_body,
        grid=(GRID,),
        in_specs=[
            pl.BlockSpec((NC, BR, D_H), lambda i: (0, i, 0)),
            pl.BlockSpec((BR, D_H), lambda i: (i, 0)),
            pl.BlockSpec((BR, 1), lambda i: (i, 0)),
            pl.BlockSpec((1, D_OUT), lambda i: (0, 0)),
        ],
        out_specs=pl.BlockSpec((BR, D_OUT), lambda i: (i, 0)),
        out_shape=jax.ShapeDtypeStruct((N, D_OUT), jnp.float32),
    )


def kernel(x, edge_index, W1, b1, g1, be1, W2, b2, g2, be2, W3, b3):
    src = edge_index[0]
    dst = edge_index[1]
    zeros_h = jnp.zeros((N, D_H), jnp.float32)
    W3p = jnp.pad(W3, ((0, 0), (0, D_H - D_OUT)))

    degp = _build_degree()(dst)
    degT = degp.reshape(NC, NP)[:, :N].T
    y1, dinv = _build_tc1()(x, W1, degT)

    agg1 = _build_aggregate(D_H)(y1, src, dst, zeros_h)
    y2 = _build_tcmid(D_H)(agg1, y1, dinv, b1.reshape(1, -1),
                           g1.reshape(1, -1), be1.reshape(1, -1), W2)

    agg2 = _build_aggregate(D_H)(y2, src, dst, zeros_h)
    y3 = _build_tcmid(D_H)(agg2, y2, dinv, b2.reshape(1, -1),
                           g2.reshape(1, -1), be2.reshape(1, -1), W3p)

    agg3 = _build_aggregate(D_H)(y3, src, dst, zeros_h)
    return _build_tcf()(agg3, y3, dinv, b3.reshape(1, -1))
